# baseline (speedup 1.0000x reference)
import functools

import jax
import jax.numpy as jnp
from jax import lax
from jax.experimental import pallas as pl
from jax.experimental.pallas import tpu as pltpu
from jax.experimental.pallas import tpu_sc as plsc

_N = 10000
_E = 320000
_D = 128
_DH = _D // 2

_NC = 2
_NS = 16
_L = 16
_CH = 100
_ECHUNKS = _E // _CH
_TCHUNKS = _ECHUNKS // _NS
_RPT = 632
_RLAST = _N - 15 * _RPT
_NB = 2


def _mesh():
    return plsc.VectorSubcoreMesh(
        core_axis_name="c", subcore_axis_name="s",
        num_cores=_NC, num_subcores=_NS)


@functools.partial(
    pl.kernel,
    out_type=jax.ShapeDtypeStruct((_N, _D), jnp.float32),
    mesh=_mesh(),
    compiler_params=pltpu.CompilerParams(use_tc_tiling_on_sc=False),
    scratch_types=[
        pltpu.VMEM((_TCHUNKS, _CH), jnp.int32),
        pltpu.VMEM((_TCHUNKS, _CH), jnp.int32),
        pltpu.VMEM((_TCHUNKS, _CH), jnp.float32),
        [pltpu.VMEM((_CH, _DH), jnp.float32)] * _NB,
        [pltpu.VMEM((_CH, _DH), jnp.float32)] * _NB,
        pltpu.VMEM_SHARED((_N, _DH), jnp.float32),
        pltpu.SemaphoreType.DMA,
        pltpu.SemaphoreType.DMA,
        pltpu.SemaphoreType.DMA,
        [pltpu.SemaphoreType.DMA] * _NB,
        [pltpu.SemaphoreType.DMA] * _NB,
    ],
)
def _aggregate(s_hbm, srcadj_hbm, dst_hbm, norm_hbm, zeros_hbm, out_hbm,
               src_v, dst_v, norm_v, rows, scaled, z_sh,
               sem_si, sem_di, sem_ni, gs, ss):
    c = lax.axis_index("c")
    s = lax.axis_index("s")

    srow = pl.multiple_of((c * _NS + s) * _TCHUNKS, 2)
    drow = pl.multiple_of(s * _TCHUNKS, 2)
    a_s = pltpu.async_copy(srcadj_hbm.at[pl.ds(srow, _TCHUNKS)], src_v, sem_si)
    a_d = pltpu.async_copy(dst_hbm.at[pl.ds(drow, _TCHUNKS)], dst_v, sem_di)
    a_n = pltpu.async_copy(norm_hbm.at[pl.ds(drow, _TCHUNKS)], norm_v, sem_ni)

    rbase = pl.multiple_of(s * _RPT, 8)

    @pl.when(s < _NS - 1)
    def _zero_main():
        pltpu.sync_copy(zeros_hbm.at[pl.ds(rbase, _RPT)],
                        z_sh.at[pl.ds(rbase, _RPT)])

    @pl.when(s == _NS - 1)
    def _zero_last():
        pltpu.sync_copy(zeros_hbm.at[pl.ds(15 * _RPT, _RLAST)],
                        z_sh.at[pl.ds(15 * _RPT, _RLAST)])
    plsc.subcore_barrier()
    a_s.wait()
    a_d.wait()
    a_n.wait()

    for b in range(_NB):
        pltpu.async_copy(s_hbm.at[src_v.at[b]], rows[b], gs[b])

    def _scale(t, rows_ref, out_ref):
        def _grp(g, c2):
            nv16 = norm_v[t, pl.ds(g * _L, _L)]
            for l in range(_L):
                n = nv16[l]
                e = g * _L + l
                vals = [rows_ref[e, pl.ds(j * _L, _L)] * n
                        for j in range(_DH // _L)]
                for j in range(_DH // _L):
                    out_ref[e, pl.ds(j * _L, _L)] = vals[j]
            return c2
        lax.fori_loop(0, _CH // _L, _grp, 0)

    def _round(i, carry):
        for b in range(_NB):
            t = _NB * i + b

            @pl.when(t < _TCHUNKS)
            def _step():
                pltpu.make_async_copy(
                    s_hbm.at[src_v.at[t]], rows[b], gs[b]).wait()

                @pl.when(t >= _NB)
                def _drain_scatter():
                    pltpu.make_async_copy(
                        scaled[b], z_sh.at[dst_v.at[t]], ss[b]).wait()

                _scale(t, rows[b], scaled[b])

                @pl.when(t + _NB < _TCHUNKS)
                def _prefetch():
                    pltpu.async_copy(s_hbm.at[src_v.at[t + _NB]],
                                     rows[b], gs[b])

                pltpu.async_copy(scaled[b], z_sh.at[dst_v.at[t]],
                                 ss[b], add=True)
        return carry
    lax.fori_loop(0, (_TCHUNKS + _NB - 1) // _NB, _round, 0)
    for b in range(_NB):
        pltpu.make_async_copy(scaled[b], z_sh.at[dst_v.at[b]], ss[b]).wait()

    plsc.subcore_barrier()
    cbase = pl.multiple_of(c * _DH, 8)

    @pl.when(s < _NS - 1)
    def _pub_main():
        pltpu.sync_copy(z_sh.at[pl.ds(rbase, _RPT)],
                        out_hbm.at[pl.ds(rbase, _RPT), pl.ds(cbase, _DH)])

    @pl.when(s == _NS - 1)
    def _pub_last():
        pltpu.sync_copy(z_sh.at[pl.ds(15 * _RPT, _RLAST)],
                        out_hbm.at[pl.ds(15 * _RPT, _RLAST), pl.ds(cbase, _DH)])


_BM = 2000


def _mm_body(x_ref, w_ref, o_ref):
    o_ref[...] = jnp.dot(x_ref[...], w_ref[...],
                         preferred_element_type=jnp.float32)


def _matmul(x, wt):
    return pl.pallas_call(
        _mm_body,
        grid=(_N // _BM,),
        in_specs=[pl.BlockSpec((_BM, _D), lambda i: (i, 0)),
                  pl.BlockSpec((_D, _D), lambda i: (0, 0))],
        out_specs=pl.BlockSpec((_BM, _D), lambda i: (i, 0)),
        out_shape=jax.ShapeDtypeStruct((_N, _D), jnp.float32),
    )(x, wt)


def _fuse_body(p_ref, w_ref, h_ref, s_ref):
    h = jnp.maximum(p_ref[...], 0.0)
    h_ref[...] = h
    s_ref[...] = jnp.dot(h, w_ref[...], preferred_element_type=jnp.float32)


def _fuse(p, wt):
    return pl.pallas_call(
        _fuse_body,
        grid=(_N // _BM,),
        in_specs=[pl.BlockSpec((_BM, _D), lambda i: (i, 0)),
                  pl.BlockSpec((_D, _D), lambda i: (0, 0))],
        out_specs=[pl.BlockSpec((_BM, _D), lambda i: (i, 0)),
                   pl.BlockSpec((_BM, _D), lambda i: (i, 0))],
        out_shape=[jax.ShapeDtypeStruct((_N, _D), jnp.float32),
                   jax.ShapeDtypeStruct((_N, _D), jnp.float32)],
    )(p, wt)


def kernel(X, src, dst, norm, W0, W1):
    src = src.astype(jnp.int32)
    dst = dst.astype(jnp.int32)
    src2 = src * 2
    src_adj = jnp.concatenate([src2, src2 + 1]).reshape(2 * _ECHUNKS, _CH)
    dst2 = dst.reshape(_ECHUNKS, _CH)
    norm2 = norm.reshape(_ECHUNKS, _CH)
    zeros = jnp.zeros((_N, _DH), jnp.float32)
    S0 = _matmul(X, W0.T)
    P0 = _aggregate(S0.reshape(2 * _N, _DH), src_adj, dst2, norm2, zeros)
    H, S1 = _fuse(P0, W1.T)
    Z = _aggregate(S1.reshape(2 * _N, _DH), src_adj, dst2, norm2, zeros)
    return (Z, H)


# speedup vs baseline: 1.1762x; 1.1762x over previous
import functools

import jax
import jax.numpy as jnp
from jax import lax
from jax.experimental import pallas as pl
from jax.experimental.pallas import tpu as pltpu
from jax.experimental.pallas import tpu_sc as plsc

_N = 10000
_E = 320000
_D = 128
_DH = _D // 2

_NC = 2
_NS = 16
_L = 16
_CH = 80
_ECHUNKS = _E // _CH
_TCHUNKS = _ECHUNKS // _NS
_RPT = 632
_RLAST = _N - 15 * _RPT
_NB = 3


def _mesh():
    return plsc.VectorSubcoreMesh(
        core_axis_name="c", subcore_axis_name="s",
        num_cores=_NC, num_subcores=_NS)


@functools.partial(
    pl.kernel,
    out_type=jax.ShapeDtypeStruct((_N, _D), jnp.float32),
    mesh=_mesh(),
    compiler_params=pltpu.CompilerParams(use_tc_tiling_on_sc=False),
    scratch_types=[
        pltpu.VMEM((_TCHUNKS, _CH), jnp.int32),
        pltpu.VMEM((_TCHUNKS, _CH), jnp.int32),
        pltpu.VMEM((_TCHUNKS, _CH), jnp.float32),
        [pltpu.VMEM((_CH, _DH), jnp.float32)] * _NB,
        [pltpu.VMEM((_CH, _DH), jnp.float32)] * _NB,
        pltpu.VMEM_SHARED((_N, _DH), jnp.float32),
        pltpu.SemaphoreType.DMA,
        pltpu.SemaphoreType.DMA,
        pltpu.SemaphoreType.DMA,
        [pltpu.SemaphoreType.DMA] * _NB,
        [pltpu.SemaphoreType.DMA] * _NB,
    ],
)
def _aggregate(s_hbm, srcadj_hbm, dst_hbm, norm_hbm, zeros_hbm, out_hbm,
               src_v, dst_v, norm_v, rows, scaled, z_sh,
               sem_si, sem_di, sem_ni, gs, ss):
    c = lax.axis_index("c")
    s = lax.axis_index("s")

    srow = pl.multiple_of((c * _NS + s) * _TCHUNKS, 2)
    drow = pl.multiple_of(s * _TCHUNKS, 2)
    a_s = pltpu.async_copy(srcadj_hbm.at[pl.ds(srow, _TCHUNKS)], src_v, sem_si)
    a_d = pltpu.async_copy(dst_hbm.at[pl.ds(drow, _TCHUNKS)], dst_v, sem_di)
    a_n = pltpu.async_copy(norm_hbm.at[pl.ds(drow, _TCHUNKS)], norm_v, sem_ni)

    rbase = pl.multiple_of(s * _RPT, 8)

    @pl.when(s < _NS - 1)
    def _zero_main():
        pltpu.sync_copy(zeros_hbm.at[pl.ds(rbase, _RPT)],
                        z_sh.at[pl.ds(rbase, _RPT)])

    @pl.when(s == _NS - 1)
    def _zero_last():
        pltpu.sync_copy(zeros_hbm.at[pl.ds(15 * _RPT, _RLAST)],
                        z_sh.at[pl.ds(15 * _RPT, _RLAST)])
    plsc.subcore_barrier()
    a_s.wait()
    a_d.wait()
    a_n.wait()

    for b in range(_NB):
        pltpu.async_copy(s_hbm.at[src_v.at[b]], rows[b], gs[b])

    def _scale(t, rows_ref, out_ref):
        def _grp(g, c2):
            nv16 = norm_v[t, pl.ds(g * _L, _L)]
            for l in range(_L):
                n = nv16[l]
                e = g * _L + l
                vals = [rows_ref[e, pl.ds(j * _L, _L)] * n
                        for j in range(_DH // _L)]
                for j in range(_DH // _L):
                    out_ref[e, pl.ds(j * _L, _L)] = vals[j]
            return c2
        lax.fori_loop(0, _CH // _L, _grp, 0)

    def _round(i, carry):
        for b in range(_NB):
            t = _NB * i + b

            @pl.when(t < _TCHUNKS)
            def _step():
                pltpu.make_async_copy(
                    s_hbm.at[src_v.at[t]], rows[b], gs[b]).wait()

                @pl.when(t >= _NB)
                def _drain_scatter():
                    pltpu.make_async_copy(
                        scaled[b], z_sh.at[dst_v.at[t]], ss[b]).wait()

                _scale(t, rows[b], scaled[b])

                @pl.when(t + _NB < _TCHUNKS)
                def _prefetch():
                    pltpu.async_copy(s_hbm.at[src_v.at[t + _NB]],
                                     rows[b], gs[b])

                pltpu.async_copy(scaled[b], z_sh.at[dst_v.at[t]],
                                 ss[b], add=True)
        return carry
    lax.fori_loop(0, (_TCHUNKS + _NB - 1) // _NB, _round, 0)
    for b in range(_NB):
        pltpu.make_async_copy(scaled[b], z_sh.at[dst_v.at[b]], ss[b]).wait()

    plsc.subcore_barrier()
    cbase = pl.multiple_of(c * _DH, 8)

    @pl.when(s < _NS - 1)
    def _pub_main():
        pltpu.sync_copy(z_sh.at[pl.ds(rbase, _RPT)],
                        out_hbm.at[pl.ds(rbase, _RPT), pl.ds(cbase, _DH)])

    @pl.when(s == _NS - 1)
    def _pub_last():
        pltpu.sync_copy(z_sh.at[pl.ds(15 * _RPT, _RLAST)],
                        out_hbm.at[pl.ds(15 * _RPT, _RLAST), pl.ds(cbase, _DH)])


_BM = 2000


def _mm_body(x_ref, w_ref, o_ref):
    o_ref[...] = jnp.dot(x_ref[...], w_ref[...],
                         preferred_element_type=jnp.float32)


def _matmul(x, wt):
    return pl.pallas_call(
        _mm_body,
        grid=(_N // _BM,),
        in_specs=[pl.BlockSpec((_BM, _D), lambda i: (i, 0)),
                  pl.BlockSpec((_D, _D), lambda i: (0, 0))],
        out_specs=pl.BlockSpec((_BM, _D), lambda i: (i, 0)),
        out_shape=jax.ShapeDtypeStruct((_N, _D), jnp.float32),
    )(x, wt)


def _fuse_body(p_ref, w_ref, h_ref, s_ref):
    h = jnp.maximum(p_ref[...], 0.0)
    h_ref[...] = h
    s_ref[...] = jnp.dot(h, w_ref[...], preferred_element_type=jnp.float32)


def _fuse(p, wt):
    return pl.pallas_call(
        _fuse_body,
        grid=(_N // _BM,),
        in_specs=[pl.BlockSpec((_BM, _D), lambda i: (i, 0)),
                  pl.BlockSpec((_D, _D), lambda i: (0, 0))],
        out_specs=[pl.BlockSpec((_BM, _D), lambda i: (i, 0)),
                   pl.BlockSpec((_BM, _D), lambda i: (i, 0))],
        out_shape=[jax.ShapeDtypeStruct((_N, _D), jnp.float32),
                   jax.ShapeDtypeStruct((_N, _D), jnp.float32)],
    )(p, wt)


def kernel(X, src, dst, norm, W0, W1):
    src = src.astype(jnp.int32)
    dst = dst.astype(jnp.int32)
    src2 = src * 2
    src_adj = jnp.concatenate([src2, src2 + 1]).reshape(2 * _ECHUNKS, _CH)
    dst2 = dst.reshape(_ECHUNKS, _CH)
    norm2 = norm.reshape(_ECHUNKS, _CH)
    zeros = jnp.zeros((_N, _DH), jnp.float32)
    S0 = _matmul(X, W0.T)
    P0 = _aggregate(S0.reshape(2 * _N, _DH), src_adj, dst2, norm2, zeros)
    H, S1 = _fuse(P0, W1.T)
    Z = _aggregate(S1.reshape(2 * _N, _DH), src_adj, dst2, norm2, zeros)
    return (Z, H)
